# Initial kernel scaffold; baseline (speedup 1.0000x reference)
import jax
import jax.numpy as jnp
from jax.experimental import pallas as pl


def kernel(x, table):
    raise NotImplementedError("write your pallas kernel here")


# speedup vs baseline: 1.2836x; 1.2836x over previous
import functools

import jax
import jax.numpy as jnp
from jax import lax
from jax.experimental import pallas as pl
from jax.experimental.pallas import tpu as pltpu
from jax.experimental.pallas import tpu_sc as plsc

BATCH = 16384
HIST = 50
DIM = 32
TOTAL = BATCH * HIST
NUM_WORKERS = 32
BLK = 128
CH_BLKS = 8
CHUNK = BLK * CH_BLKS
PER_W = TOTAL // NUM_WORKERS
N_CHUNKS = PER_W // CHUNK
N_BLOCKS = TOTAL // BLK

_mesh = plsc.VectorSubcoreMesh(core_axis_name="c", subcore_axis_name="s")


@functools.partial(
    pl.kernel,
    mesh=_mesh,
    out_type=jax.ShapeDtypeStruct((N_BLOCKS, BLK, DIM), jnp.float32),
    scratch_types=[
        pltpu.VMEM((CH_BLKS, BLK), jnp.int32),
        pltpu.VMEM((CH_BLKS, BLK, DIM), jnp.float32),
        pltpu.SemaphoreType.DMA,
    ],
    compiler_params=pltpu.CompilerParams(use_tc_tiling_on_sc=False),
)
def _sc_gather(idx_hbm, table_hbm, out_hbm, idx_v, rows_v, sem):
    wid = lax.axis_index("s") * 2 + lax.axis_index("c")
    first_chunk = wid * N_CHUNKS

    def chunk_body(c, carry):
        blk0 = (first_chunk + c) * CH_BLKS
        pltpu.sync_copy(idx_hbm.at[pl.ds(blk0, CH_BLKS)], idx_v)
        copies = [
            pltpu.async_copy(table_hbm.at[idx_v.at[j]], rows_v.at[j], sem)
            for j in range(CH_BLKS)
        ]
        for cp in copies:
            cp.wait()
        pltpu.sync_copy(rows_v, out_hbm.at[pl.ds(blk0, CH_BLKS)])
        return carry

    lax.fori_loop(0, N_CHUNKS, chunk_body, 0)


def kernel(x, table):
    idx = x.reshape(N_BLOCKS, BLK).astype(jnp.int32)
    out = _sc_gather(idx, table)
    return out.reshape(BATCH, HIST, DIM)


# speedup vs baseline: 1.3015x; 1.0140x over previous
import functools

import jax
import jax.numpy as jnp
from jax import lax
from jax.experimental import pallas as pl
from jax.experimental.pallas import tpu as pltpu
from jax.experimental.pallas import tpu_sc as plsc

BATCH = 16384
HIST = 50
DIM = 32
TOTAL = BATCH * HIST
NUM_WORKERS = 32
BLK = 128
CH_BLKS = 10
PER_W = TOTAL // NUM_WORKERS
N_CHUNKS = PER_W // (BLK * CH_BLKS)
N_BLOCKS = TOTAL // BLK

_mesh = plsc.VectorSubcoreMesh(core_axis_name="c", subcore_axis_name="s")


@functools.partial(
    pl.kernel,
    mesh=_mesh,
    out_type=jax.ShapeDtypeStruct((N_BLOCKS, BLK, DIM), jnp.float32),
    scratch_types=[
        pltpu.VMEM((2, CH_BLKS, BLK), jnp.int32),
        pltpu.VMEM((2, CH_BLKS, BLK, DIM), jnp.float32),
        pltpu.SemaphoreType.DMA,
        pltpu.SemaphoreType.DMA,
        pltpu.SemaphoreType.DMA,
        pltpu.SemaphoreType.DMA,
    ],
    compiler_params=pltpu.CompilerParams(use_tc_tiling_on_sc=False),
)
def _sc_gather(idx_hbm, table_hbm, out_hbm, idx_v, rows_v, g0, g1, w0, w1):
    gsem = (g0, g1)
    wsem = (w0, w1)
    wid = lax.axis_index("s") * 2 + lax.axis_index("c")
    first = wid * N_CHUNKS

    def load_and_fire(c, b):
        blk0 = (first + c) * CH_BLKS
        pltpu.sync_copy(idx_hbm.at[pl.ds(blk0, CH_BLKS)], idx_v.at[b])
        for j in range(CH_BLKS):
            pltpu.async_copy(
                table_hbm.at[idx_v.at[b].at[j]], rows_v.at[b].at[j], gsem[b]
            )

    def wait_rows(b, sem):
        pltpu.make_async_copy(
            out_hbm.at[pl.ds(0, CH_BLKS)], rows_v.at[b], sem
        ).wait()

    load_and_fire(0, 0)
    load_and_fire(1, 1)

    def pair_body(g, carry):
        for b in range(2):
            c = 2 * g + b
            blk0 = (first + c) * CH_BLKS
            wait_rows(b, gsem[b])
            pltpu.async_copy(rows_v.at[b], out_hbm.at[pl.ds(blk0, CH_BLKS)], wsem[b])
            wait_rows(b, wsem[b])

            @pl.when(c + 2 < N_CHUNKS)
            def _():
                load_and_fire(c + 2, b)

        return carry

    lax.fori_loop(0, N_CHUNKS // 2, pair_body, 0)


def kernel(x, table):
    idx = x.reshape(N_BLOCKS, BLK).astype(jnp.int32)
    out = _sc_gather(idx, table)
    return out.reshape(BATCH, HIST, DIM)


# speedup vs baseline: 1.7567x; 1.3498x over previous
import functools

import jax
import jax.numpy as jnp
from jax import lax
from jax.experimental import pallas as pl
from jax.experimental.pallas import tpu as pltpu
from jax.experimental.pallas import tpu_sc as plsc

BATCH = 16384
HIST = 50
DIM = 32
NUM_WORKERS = 32
ROWS_W = BATCH // NUM_WORKERS
CH = 8
N_CHUNKS = ROWS_W // CH

_mesh = plsc.VectorSubcoreMesh(core_axis_name="c", subcore_axis_name="s")


@functools.partial(
    pl.kernel,
    mesh=_mesh,
    out_type=jax.ShapeDtypeStruct((BATCH, HIST, DIM), jnp.float32),
    scratch_types=[
        pltpu.VMEM((2, CH, HIST), jnp.int32),
        pltpu.VMEM((2, CH, HIST, DIM), jnp.float32),
        pltpu.SemaphoreType.DMA,
        pltpu.SemaphoreType.DMA,
        pltpu.SemaphoreType.DMA,
        pltpu.SemaphoreType.DMA,
    ],
    compiler_params=pltpu.CompilerParams(use_tc_tiling_on_sc=False),
)
def _sc_gather(idx_hbm, table_hbm, out_hbm, idx_v, rows_v, g0, g1, w0, w1):
    gsem = (g0, g1)
    wsem = (w0, w1)
    wid = lax.axis_index("s") * 2 + lax.axis_index("c")
    first = wid * N_CHUNKS

    def load_and_fire(c, b):
        r0 = (first + c) * CH
        pltpu.sync_copy(idx_hbm.at[pl.ds(r0, CH)], idx_v.at[b])
        for j in range(CH):
            pltpu.async_copy(
                table_hbm.at[idx_v.at[b].at[j]], rows_v.at[b].at[j], gsem[b]
            )

    def wait_rows(b, sem):
        pltpu.make_async_copy(
            out_hbm.at[pl.ds(0, CH)], rows_v.at[b], sem
        ).wait()

    load_and_fire(0, 0)
    load_and_fire(1, 1)

    def pair_body(g, carry):
        for b in range(2):
            c = 2 * g + b
            r0 = (first + c) * CH
            wait_rows(b, gsem[b])
            pltpu.async_copy(rows_v.at[b], out_hbm.at[pl.ds(r0, CH)], wsem[b])
            wait_rows(b, wsem[b])

            @pl.when(c + 2 < N_CHUNKS)
            def _():
                load_and_fire(c + 2, b)

        return carry

    lax.fori_loop(0, N_CHUNKS // 2, pair_body, 0)


def kernel(x, table):
    return _sc_gather(x.astype(jnp.int32), table)


# speedup vs baseline: 1.8155x; 1.0335x over previous
import functools

import jax
import jax.numpy as jnp
from jax import lax
from jax.experimental import pallas as pl
from jax.experimental.pallas import tpu as pltpu
from jax.experimental.pallas import tpu_sc as plsc

BATCH = 16384
HIST = 50
DIM = 32
NUM_WORKERS = 32
BW = BATCH // NUM_WORKERS
LANES = 16

_mesh = plsc.VectorSubcoreMesh(core_axis_name="c", subcore_axis_name="s")


@functools.partial(
    pl.kernel,
    mesh=_mesh,
    out_type=jax.ShapeDtypeStruct((HIST, 4, 8 * BATCH), jnp.float32),
    scratch_types=[
        pltpu.VMEM((2, BW), jnp.int32),
        pltpu.VMEM((2, 4, 128, DIM), jnp.float32),
        pltpu.VMEM((2, 4 * 4096), jnp.float32),
        pltpu.SemaphoreType.DMA,
        pltpu.SemaphoreType.DMA,
        pltpu.SemaphoreType.DMA,
        pltpu.SemaphoreType.DMA,
    ],
    compiler_params=pltpu.CompilerParams(
        use_tc_tiling_on_sc=False, needs_layout_passes=False
    ),
)
def _sc_gather(xt_hbm, table_hbm, out_hbm, idx_v, rows_v, tile_v, g0, g1, w0, w1):
    gsem = (g0, g1)
    wsem = (w0, w1)
    wid = lax.axis_index("s") * 2 + lax.axis_index("c")
    b0 = wid * BW

    lane = lax.iota(jnp.int32, LANES)
    perm0 = ((lane >> 3) << 12) + ((lane & 7) << 7)
    perm1 = perm0 + 8192

    def load_and_fire(h, b):
        pltpu.sync_copy(xt_hbm.at[h, pl.ds(b0, BW)], idx_v.at[b])
        for cc in range(4):
            pltpu.async_copy(
                table_hbm.at[idx_v.at[b, pl.ds(cc * 128, 128)]],
                rows_v.at[b, cc],
                gsem[b],
            )

    def wait_gather(b):
        for cc in range(4):
            pltpu.make_async_copy(
                table_hbm.at[pl.ds(0, 128)], rows_v.at[b, cc], gsem[b]
            ).wait()

    def wait_writeback(b):
        for r in range(4):
            pltpu.make_async_copy(
                out_hbm.at[0, 0, pl.ds(0, 4096)],
                tile_v.at[b, pl.ds(r * 4096, 4096)],
                wsem[b],
            ).wait()

    load_and_fire(0, 0)
    load_and_fire(1, 1)

    def pair_body(g, carry):
        for b in range(2):
            h = 2 * g + b
            wait_gather(b)

            @pl.when(g >= 1)
            def _():
                wait_writeback(b)

            def bc_body(it, carry2):
                for k in range(4):
                    bc = 4 * it + k
                    for cc in range(4):
                        base = cc * 1024 + bc
                        v0 = rows_v[b, cc, bc, pl.ds(0, LANES)]
                        v1 = rows_v[b, cc, bc, pl.ds(LANES, LANES)]
                        plsc.store_scatter(tile_v.at[b], [perm0 + base], v0)
                        plsc.store_scatter(tile_v.at[b], [perm1 + base], v1)
                return carry2

            lax.fori_loop(0, 32, bc_body, 0)

            for r in range(4):
                pltpu.async_copy(
                    tile_v.at[b, pl.ds(r * 4096, 4096)],
                    out_hbm.at[h, r, pl.ds(wid * 4096, 4096)],
                    wsem[b],
                )

            @pl.when(h + 2 < HIST)
            def _():
                load_and_fire(h + 2, b)

        return carry

    lax.fori_loop(0, HIST // 2, pair_body, 0)
    wait_writeback(0)
    wait_writeback(1)


def kernel(x, table):
    xt = x.T.astype(jnp.int32)
    out6 = _sc_gather(xt, table).reshape(HIST, 4, 128, 8, 128)
    return out6.transpose(2, 4, 0, 1, 3).reshape(BATCH, HIST, DIM)
